# baseline (speedup 1.0000x reference)
import jax
import jax.numpy as jnp
from jax.experimental import pallas as pl

BATCH = 32
SEQ_LEN = 8192
N_ALLELES = 4
D_MODEL = 64
S_TILE = 2048
S_TILES = SEQ_LEN // S_TILE


def _body(x_ref, a_ref, p_ref, o_ref):
    emb = jax.lax.dot_general(
        x_ref[...], a_ref[...],
        dimension_numbers=(((1,), (0,)), ((), ())),
        preferred_element_type=jnp.float32,
    )
    o_ref[...] = emb + p_ref[...]


def kernel(x, allele_embedding, position_table):
    xf = x.reshape(BATCH * SEQ_LEN, N_ALLELES)
    out = pl.pallas_call(
        _body,
        grid=(S_TILES, BATCH),
        in_specs=[
            pl.BlockSpec((S_TILE, N_ALLELES), lambda s, b: (b * S_TILES + s, 0)),
            pl.BlockSpec((N_ALLELES, D_MODEL), lambda s, b: (0, 0)),
            pl.BlockSpec((S_TILE, D_MODEL), lambda s, b: (s, 0)),
        ],
        out_specs=pl.BlockSpec((S_TILE, D_MODEL), lambda s, b: (b * S_TILES + s, 0)),
        out_shape=jax.ShapeDtypeStruct((BATCH * SEQ_LEN, D_MODEL), jnp.float32),
    )(xf, allele_embedding, position_table)
    return out.reshape(BATCH, SEQ_LEN, D_MODEL)
